# baseline (speedup 1.0000x reference)
import functools

import jax
import jax.numpy as jnp
from jax import lax
from jax.experimental import pallas as pl
from jax.experimental.pallas import tpu as pltpu
from jax.experimental.pallas import tpu_sc as plsc

_NUM_STOCKS = 100000
_EMBED_DIM = 32
_BATCH = 16384
_SEQ_LEN = 50

_B = _BATCH * _SEQ_LEN
_NC = 2
_NS = 16
_NW = _NC * _NS
_BW = _BATCH // _NW
_IDXW = 128
_NSTREAM = _BW // _IDXW

_mesh = plsc.VectorSubcoreMesh(core_axis_name="c", subcore_axis_name="s")


@functools.partial(
    pl.kernel,
    mesh=_mesh,
    out_type=jax.ShapeDtypeStruct((_SEQ_LEN, _EMBED_DIM, _BATCH), jnp.float32),
    scratch_types=[
        pltpu.VMEM((2, _BW), jnp.int32),
        pltpu.VMEM((2, _BW, _EMBED_DIM), jnp.float32),
        pltpu.VMEM((2, _EMBED_DIM, _BW), jnp.float32),
        pltpu.SemaphoreType.DMA((2,)),
        pltpu.SemaphoreType.DMA((2,)),
        pltpu.SemaphoreType.DMA((2,)),
    ],
    compiler_params=pltpu.CompilerParams(
        use_tc_tiling_on_sc=False, needs_layout_passes=False
    ),
)
def _emb_lookup(idx_hbm, table_hbm, out_hbm, il_v, rows_v, tr_v,
                isem, gsem, osem):
    wid = lax.axis_index("s") * _NC + lax.axis_index("c")
    b0 = wid * _BW

    iota16 = lax.iota(jnp.int32, 16)

    def load_ilist(p, s):
        pltpu.async_copy(
            idx_hbm.at[pl.ds(s * _BATCH + b0, _BW)], il_v.at[p], isem.at[p]
        )

    def wait_ilist(p):
        pltpu.make_async_copy(
            idx_hbm.at[pl.ds(b0, _BW)], il_v.at[p], isem.at[p]
        ).wait()

    def fire_gathers(p):
        for j in range(_NSTREAM):
            pltpu.async_copy(
                table_hbm.at[il_v.at[p].at[pl.ds(j * _IDXW, _IDXW)]],
                rows_v.at[p].at[pl.ds(j * _IDXW, _IDXW)],
                gsem.at[p],
            )

    def drain_gathers(p):
        for j in range(_NSTREAM):
            pltpu.make_async_copy(
                table_hbm.at[il_v.at[p].at[pl.ds(j * _IDXW, _IDXW)]],
                rows_v.at[p].at[pl.ds(j * _IDXW, _IDXW)],
                gsem.at[p],
            ).wait()

    def transpose(p):
        @plsc.parallel_loop(0, _EMBED_DIM, step=1, unroll=8)
        def dbody(d):
            col = jnp.full((16,), d, jnp.int32)
            for bb in range(_BW // 16):
                rowi = iota16 + bb * 16
                tr_v[p, d, pl.ds(bb * 16, 16)] = plsc.load_gather(
                    rows_v.at[p], [rowi, col]
                )

    load_ilist(0, 0)
    wait_ilist(0)
    fire_gathers(0)
    load_ilist(1, 1)

    def outer(it, carry):
        for p in range(2):
            s = it * 2 + p
            @pl.when(s + 1 < _SEQ_LEN)
            def _():
                wait_ilist(1 - p)
                fire_gathers(1 - p)

            drain_gathers(p)

            @pl.when(s + 2 < _SEQ_LEN)
            def _():
                load_ilist(p, s + 2)

            @pl.when(s >= 2)
            def _():
                pltpu.make_async_copy(
                    tr_v.at[p],
                    out_hbm.at[0].at[:, pl.ds(b0, _BW)],
                    osem.at[p],
                ).wait()

            transpose(p)
            pltpu.async_copy(
                tr_v.at[p], out_hbm.at[s].at[:, pl.ds(b0, _BW)], osem.at[p]
            )
        return carry

    lax.fori_loop(0, _SEQ_LEN // 2, outer, 0)

    for p in range(2):
        pltpu.make_async_copy(
            tr_v.at[p], out_hbm.at[0].at[:, pl.ds(b0, _BW)], osem.at[p]
        ).wait()


def kernel(stock_ids, weight):
    idx_t = stock_ids.T.reshape(_B)
    out3 = _emb_lookup(idx_t, weight)
    return out3.transpose(2, 0, 1)


# speedup vs baseline: 1.0975x; 1.0975x over previous
import functools

import jax
import jax.numpy as jnp
from jax import lax
from jax.experimental import pallas as pl
from jax.experimental.pallas import tpu as pltpu
from jax.experimental.pallas import tpu_sc as plsc

_NUM_STOCKS = 100000
_EMBED_DIM = 32
_BATCH = 16384
_SEQ_LEN = 50

_B = _BATCH * _SEQ_LEN
_NC = 2
_NS = 16
_NW = _NC * _NS
_BW = _BATCH // _NW
_IDXW = 128
_NSTREAM = _BW // _IDXW

_mesh = plsc.VectorSubcoreMesh(core_axis_name="c", subcore_axis_name="s")


@functools.partial(
    pl.kernel,
    mesh=_mesh,
    out_type=jax.ShapeDtypeStruct((_SEQ_LEN, _EMBED_DIM, _BATCH), jnp.float32),
    scratch_types=[
        pltpu.VMEM((_BW * _SEQ_LEN,), jnp.int32),
        pltpu.VMEM((2, _BW), jnp.int32),
        pltpu.VMEM((2, _BW, _EMBED_DIM), jnp.float32),
        pltpu.VMEM((2, _EMBED_DIM, _BW), jnp.float32),
        pltpu.SemaphoreType.DMA,
        pltpu.SemaphoreType.DMA((2,)),
        pltpu.SemaphoreType.DMA((2,)),
    ],
    compiler_params=pltpu.CompilerParams(
        use_tc_tiling_on_sc=False, needs_layout_passes=False
    ),
)
def _emb_lookup(idx_hbm, table_hbm, out_hbm, idx_v, il_v, rows_v, tr_v,
                isem, gsem, osem):
    wid = lax.axis_index("s") * _NC + lax.axis_index("c")
    b0 = wid * _BW
    pltpu.async_copy(
        idx_hbm.at[pl.ds(b0 * _SEQ_LEN, _BW * _SEQ_LEN)], idx_v, isem
    ).wait()

    iota16 = lax.iota(jnp.int32, 16)
    iota_s = iota16 * _SEQ_LEN

    def build_ilist(p, s):
        @plsc.parallel_loop(0, _BW // 16, step=1, unroll=8)
        def bbody(bb):
            addr = iota_s + (bb * 16 * _SEQ_LEN + s)
            il_v[p, pl.ds(bb * 16, 16)] = plsc.load_gather(idx_v, [addr])

    def fire_gathers(p):
        for j in range(_NSTREAM):
            pltpu.async_copy(
                table_hbm.at[il_v.at[p].at[pl.ds(j * _IDXW, _IDXW)]],
                rows_v.at[p].at[pl.ds(j * _IDXW, _IDXW)],
                gsem.at[p],
            )

    def drain_gathers(p):
        for j in range(_NSTREAM):
            pltpu.make_async_copy(
                table_hbm.at[il_v.at[p].at[pl.ds(j * _IDXW, _IDXW)]],
                rows_v.at[p].at[pl.ds(j * _IDXW, _IDXW)],
                gsem.at[p],
            ).wait()

    def transpose(p):
        @plsc.parallel_loop(0, _EMBED_DIM, step=1, unroll=4)
        def dbody(d):
            col = jnp.full((16,), d, jnp.int32)
            for bb in range(_BW // 16):
                rowi = iota16 + bb * 16
                tr_v[p, d, pl.ds(bb * 16, 16)] = plsc.load_gather(
                    rows_v.at[p], [rowi, col]
                )

    build_ilist(0, 0)
    fire_gathers(0)

    def outer(it, carry):
        for p in range(2):
            s = it * 2 + p
            drain_gathers(p)

            @pl.when(s + 1 < _SEQ_LEN)
            def _():
                build_ilist(1 - p, s + 1)
                fire_gathers(1 - p)

            @pl.when(s >= 2)
            def _():
                pltpu.make_async_copy(
                    tr_v.at[p],
                    out_hbm.at[0].at[:, pl.ds(b0, _BW)],
                    osem.at[p],
                ).wait()

            transpose(p)
            pltpu.async_copy(
                tr_v.at[p], out_hbm.at[s].at[:, pl.ds(b0, _BW)], osem.at[p]
            )
        return carry

    lax.fori_loop(0, _SEQ_LEN // 2, outer, 0)

    for p in range(2):
        pltpu.make_async_copy(
            tr_v.at[p], out_hbm.at[0].at[:, pl.ds(b0, _BW)], osem.at[p]
        ).wait()


def kernel(stock_ids, weight):
    idx_flat = stock_ids.reshape(_B)
    out3 = _emb_lookup(idx_flat, weight)
    return out3.transpose(2, 0, 1)


# speedup vs baseline: 2.0879x; 1.9024x over previous
import functools

import jax
import jax.numpy as jnp
from jax import lax
from jax.experimental import pallas as pl
from jax.experimental.pallas import tpu as pltpu
from jax.experimental.pallas import tpu_sc as plsc

_NUM_STOCKS = 100000
_EMBED_DIM = 32
_BATCH = 16384
_SEQ_LEN = 50

_B = _BATCH * _SEQ_LEN
_NC = 2
_NS = 16
_NW = _NC * _NS
_BW = _BATCH // _NW
_IDXW = 128
_NSTREAM = _BW // _IDXW
_TRP = _BW + 1

_mesh = plsc.VectorSubcoreMesh(core_axis_name="c", subcore_axis_name="s")


@functools.partial(
    pl.kernel,
    mesh=_mesh,
    out_type=jax.ShapeDtypeStruct((_SEQ_LEN, _EMBED_DIM, _BATCH), jnp.float32),
    scratch_types=[
        pltpu.VMEM((_BW * _SEQ_LEN,), jnp.int32),
        pltpu.VMEM((2, _BW), jnp.int32),
        pltpu.VMEM((2, _BW, _EMBED_DIM), jnp.float32),
        pltpu.VMEM((2, _EMBED_DIM, _TRP), jnp.float32),
        pltpu.SemaphoreType.DMA,
        pltpu.SemaphoreType.DMA((2,)),
        pltpu.SemaphoreType.DMA((2,)),
    ],
    compiler_params=pltpu.CompilerParams(
        use_tc_tiling_on_sc=False, needs_layout_passes=False
    ),
)
def _emb_lookup(idx_hbm, table_hbm, out_hbm, idx_v, il_v, rows_v, tr_v,
                isem, gsem, osem):
    wid = lax.axis_index("s") * _NC + lax.axis_index("c")
    b0 = wid * _BW
    pltpu.async_copy(
        idx_hbm.at[pl.ds(b0 * _SEQ_LEN, _BW * _SEQ_LEN)], idx_v, isem
    ).wait()

    iota16 = lax.iota(jnp.int32, 16)
    iota_s = iota16 * _SEQ_LEN

    def build_ilist(p, s):
        @plsc.parallel_loop(0, _BW // 16, step=1, unroll=8)
        def bbody(bb):
            addr = iota_s + (bb * 16 * _SEQ_LEN + s)
            il_v[p, pl.ds(bb * 16, 16)] = plsc.load_gather(idx_v, [addr])

    def fire_gathers(p):
        for j in range(_NSTREAM):
            pltpu.async_copy(
                table_hbm.at[il_v.at[p].at[pl.ds(j * _IDXW, _IDXW)]],
                rows_v.at[p].at[pl.ds(j * _IDXW, _IDXW)],
                gsem.at[p],
            )

    def drain_gathers(p):
        for j in range(_NSTREAM):
            pltpu.make_async_copy(
                table_hbm.at[il_v.at[p].at[pl.ds(j * _IDXW, _IDXW)]],
                rows_v.at[p].at[pl.ds(j * _IDXW, _IDXW)],
                gsem.at[p],
            ).wait()

    def transpose(p):
        @plsc.parallel_loop(0, _BW, step=1, unroll=8)
        def bbody(b):
            bvec = jnp.full((16,), b, jnp.int32)
            v0 = rows_v[p, b, pl.ds(0, 16)]
            v1 = rows_v[p, b, pl.ds(16, 16)]
            plsc.store_scatter(tr_v.at[p], [iota16, bvec], v0)
            plsc.store_scatter(tr_v.at[p], [iota16 + 16, bvec], v1)

    build_ilist(0, 0)
    fire_gathers(0)

    def outer(it, carry):
        for p in range(2):
            s = it * 2 + p
            drain_gathers(p)

            @pl.when(s + 1 < _SEQ_LEN)
            def _():
                build_ilist(1 - p, s + 1)
                fire_gathers(1 - p)

            @pl.when(s >= 2)
            def _():
                pltpu.make_async_copy(
                    tr_v.at[p].at[:, pl.ds(0, _BW)],
                    out_hbm.at[0].at[:, pl.ds(b0, _BW)],
                    osem.at[p],
                ).wait()

            transpose(p)
            pltpu.async_copy(
                tr_v.at[p].at[:, pl.ds(0, _BW)],
                out_hbm.at[s].at[:, pl.ds(b0, _BW)],
                osem.at[p],
            )
        return carry

    lax.fori_loop(0, _SEQ_LEN // 2, outer, 0)

    for p in range(2):
        pltpu.make_async_copy(
            tr_v.at[p].at[:, pl.ds(0, _BW)],
            out_hbm.at[0].at[:, pl.ds(b0, _BW)],
            osem.at[p],
        ).wait()


def kernel(stock_ids, weight):
    idx_flat = stock_ids.reshape(_B)
    out3 = _emb_lookup(idx_flat, weight)
    return out3.transpose(2, 0, 1)


# speedup vs baseline: 2.3545x; 1.1277x over previous
import functools

import jax
import jax.numpy as jnp
from jax import lax
from jax.experimental import pallas as pl
from jax.experimental.pallas import tpu as pltpu
from jax.experimental.pallas import tpu_sc as plsc

_NUM_STOCKS = 100000
_EMBED_DIM = 32
_BATCH = 16384
_SEQ_LEN = 50

_B = _BATCH * _SEQ_LEN
_NC = 2
_NS = 16
_NW = _NC * _NS
_BW = _BATCH // _NW
_IDXW = 128
_NSTREAM = _BW // _IDXW
_TRP = _BW + 1

_mesh = plsc.VectorSubcoreMesh(core_axis_name="c", subcore_axis_name="s")


@functools.partial(
    pl.kernel,
    mesh=_mesh,
    out_type=jax.ShapeDtypeStruct((_SEQ_LEN, _EMBED_DIM, _BATCH), jnp.float32),
    scratch_types=[
        pltpu.VMEM((_BW * _SEQ_LEN,), jnp.int32),
        pltpu.VMEM((2, _BW), jnp.int32),
        pltpu.VMEM((2, _BW, _EMBED_DIM), jnp.float32),
        pltpu.VMEM((2, _EMBED_DIM, _TRP), jnp.float32),
        pltpu.SemaphoreType.DMA,
        pltpu.SemaphoreType.DMA((2,)),
        pltpu.SemaphoreType.DMA((2,)),
    ],
    compiler_params=pltpu.CompilerParams(
        use_tc_tiling_on_sc=False, needs_layout_passes=False
    ),
)
def _emb_lookup(idx_hbm, table_hbm, out_hbm, idx_v, il_v, rows_v, tr_v,
                isem, gsem, osem):
    wid = lax.axis_index("s") * _NC + lax.axis_index("c")
    b0 = wid * _BW
    pltpu.async_copy(
        idx_hbm.at[pl.ds(b0 * _SEQ_LEN, _BW * _SEQ_LEN)], idx_v, isem
    ).wait()

    iota16 = lax.iota(jnp.int32, 16)
    iota_s = iota16 * _SEQ_LEN

    def build_ilist(p, s):
        @plsc.parallel_loop(0, _BW // 16, step=1, unroll=8)
        def bbody(bb):
            addr = iota_s + (bb * 16 * _SEQ_LEN + s)
            il_v[p, pl.ds(bb * 16, 16)] = plsc.load_gather(idx_v, [addr])

    def fire_gathers(p):
        for j in range(_NSTREAM):
            pltpu.async_copy(
                table_hbm.at[il_v.at[p].at[pl.ds(j * _IDXW, _IDXW)]],
                rows_v.at[p].at[pl.ds(j * _IDXW, _IDXW)],
                gsem.at[p],
            )

    def drain_gathers(p):
        for j in range(_NSTREAM):
            pltpu.make_async_copy(
                table_hbm.at[il_v.at[p].at[pl.ds(j * _IDXW, _IDXW)]],
                rows_v.at[p].at[pl.ds(j * _IDXW, _IDXW)],
                gsem.at[p],
            ).wait()

    def transpose(p):
        @plsc.parallel_loop(0, _BW, step=1, unroll=8)
        def bbody(b):
            bvec = jnp.full((16,), b, jnp.int32)
            v0 = rows_v[p, b, pl.ds(0, 16)]
            v1 = rows_v[p, b, pl.ds(16, 16)]
            plsc.store_scatter(tr_v.at[p], [iota16, bvec], v0)
            plsc.store_scatter(tr_v.at[p], [iota16 + 16, bvec], v1)

    build_ilist(0, 0)
    fire_gathers(0)

    def outer(it, carry):
        for p in range(2):
            s = it * 2 + p
            drain_gathers(p)

            @pl.when(s + 1 < _SEQ_LEN)
            def _():
                build_ilist(1 - p, s + 1)
                fire_gathers(1 - p)

            @pl.when(s >= 2)
            def _():
                pltpu.make_async_copy(
                    tr_v.at[p].at[:, pl.ds(0, _BW)],
                    out_hbm.at[0].at[:, pl.ds(b0, _BW)],
                    osem.at[p],
                ).wait()

            transpose(p)
            pltpu.async_copy(
                tr_v.at[p].at[:, pl.ds(0, _BW)],
                out_hbm.at[s].at[:, pl.ds(b0, _BW)],
                osem.at[p],
            )
        return carry

    lax.fori_loop(0, _SEQ_LEN // 2, outer, 0)

    for p in range(2):
        pltpu.make_async_copy(
            tr_v.at[p].at[:, pl.ds(0, _BW)],
            out_hbm.at[0].at[:, pl.ds(b0, _BW)],
            osem.at[p],
        ).wait()


_RQ = 4096
_RUNITS_PER_W = 25

@functools.partial(
    pl.kernel,
    mesh=_mesh,
    out_type=jax.ShapeDtypeStruct((_SEQ_LEN, _EMBED_DIM, _BATCH), jnp.float32),
    scratch_types=[
        pltpu.VMEM((2, 8, _RQ), jnp.float32),
        pltpu.SemaphoreType.DMA((2,)),
        pltpu.SemaphoreType.DMA((2,)),
    ],
    compiler_params=pltpu.CompilerParams(
        use_tc_tiling_on_sc=True, needs_layout_passes=False
    ),
)
def _retile(flat_hbm, out_hbm, buf_v, rsem, wsem):
    wid = lax.axis_index("s") * _NC + lax.axis_index("c")
    u0 = wid * _RUNITS_PER_W

    def fire_reads(p, u):
        s = u // 16
        rem = u % 16
        dt = rem // 4
        q = rem % 4
        for r in range(8):
            pltpu.async_copy(
                flat_hbm.at[
                    pl.ds(s * (_EMBED_DIM * _BATCH)
                          + (dt * 8 + r) * _BATCH + q * _RQ, _RQ)
                ],
                buf_v.at[p].at[r],
                rsem.at[p],
            )

    def wait_reads(p):
        for r in range(8):
            pltpu.make_async_copy(
                flat_hbm.at[pl.ds(0, _RQ)], buf_v.at[p].at[r], rsem.at[p]
            ).wait()

    def wait_write(p):
        pltpu.make_async_copy(
            buf_v.at[p],
            out_hbm.at[0].at[pl.ds(0, 8), pl.ds(0, _RQ)],
            wsem.at[p],
        ).wait()

    fire_reads(0, u0)

    def outer(it, carry):
        for p in range(2):
            u = it * 2 + p

            @pl.when(u < _RUNITS_PER_W)
            def _():
                ug = u0 + u
                wait_reads(p)

                @pl.when(u + 1 < _RUNITS_PER_W)
                def _():
                    @pl.when(u >= 1)
                    def _():
                        wait_write(1 - p)
                    fire_reads(1 - p, ug + 1)

                s = ug // 16
                rem = ug % 16
                dt = pl.multiple_of(rem // 4 * 8, 8)
                q = pl.multiple_of(rem % 4 * _RQ, _RQ)
                pltpu.async_copy(
                    buf_v.at[p],
                    out_hbm.at[s].at[pl.ds(dt, 8), pl.ds(q, _RQ)],
                    wsem.at[p],
                )
        return carry

    lax.fori_loop(0, (_RUNITS_PER_W + 2) // 2, outer, 0)

    wait_write(0)
    wait_write(1)


def kernel(stock_ids, weight):
    idx_flat = stock_ids.reshape(_B)
    out3 = _emb_lookup(idx_flat, weight)
    out3t = _retile(out3.reshape(_SEQ_LEN * _EMBED_DIM * _BATCH))
    return out3t.transpose(2, 0, 1)
